# Initial kernel scaffold; baseline (speedup 1.0000x reference)
import jax
import jax.numpy as jnp
from jax.experimental import pallas as pl


def kernel(tokens, embedding_weight):
    raise NotImplementedError("write your pallas kernel here")


# speedup vs baseline: 1.4968x; 1.4968x over previous
import functools

import jax
import jax.numpy as jnp
from jax import lax
from jax.experimental import pallas as pl
from jax.experimental.pallas import tpu as pltpu
from jax.experimental.pallas import tpu_sc as plsc

VOCAB = 1000000
EMBED = 32
NUM_CORES = 2
NUM_SUBCORES = 16
NUM_WORKERS = NUM_CORES * NUM_SUBCORES


@functools.partial(jax.jit, static_argnums=(2, 3))
def _gather_rows(idx, table, b_per_w, chunk):
    n_chunks = b_per_w // chunk
    B = idx.shape[0]
    mesh = plsc.VectorSubcoreMesh(core_axis_name="c", subcore_axis_name="s")

    @functools.partial(
        pl.kernel,
        mesh=mesh,
        out_type=jax.ShapeDtypeStruct((B, EMBED), jnp.float32),
        scratch_types=[
            pltpu.VMEM((chunk,), jnp.int32),
            pltpu.VMEM((chunk, EMBED), jnp.float32),
            pltpu.SemaphoreType.DMA,
        ],
        compiler_params=pltpu.CompilerParams(use_tc_tiling_on_sc=False),
    )
    def k(idx_hbm, table_hbm, out_hbm, idx_v, rows_v, sem):
        wid = lax.axis_index("s") * NUM_CORES + lax.axis_index("c")
        base = wid * b_per_w
        for c in range(n_chunks):
            off = base + c * chunk
            pltpu.sync_copy(idx_hbm.at[pl.ds(off, chunk)], idx_v)
            pltpu.async_copy(table_hbm.at[idx_v], rows_v, sem).wait()
            pltpu.sync_copy(rows_v, out_hbm.at[pl.ds(off, chunk)])

    return k(idx, table)


def kernel(tokens, embedding_weight):
    B = tokens.shape[0] * tokens.shape[1]
    idx = tokens.reshape(B).astype(jnp.int32)
    b_per_w = B // NUM_WORKERS
    out = _gather_rows(idx, embedding_weight, b_per_w, 3200)
    return out.reshape(tokens.shape + (EMBED,))


# speedup vs baseline: 1.5020x; 1.0035x over previous
import functools

import jax
import jax.numpy as jnp
from jax import lax
from jax.experimental import pallas as pl
from jax.experimental.pallas import tpu as pltpu
from jax.experimental.pallas import tpu_sc as plsc

VOCAB = 1000000
EMBED = 32
NUM_CORES = 2
NUM_SUBCORES = 16
NUM_WORKERS = NUM_CORES * NUM_SUBCORES


@functools.partial(jax.jit, static_argnums=(2, 3))
def _gather_rows(idx, table, b_per_w, chunk):
    n_chunks = b_per_w // chunk
    B = idx.shape[0]
    mesh = plsc.VectorSubcoreMesh(core_axis_name="c", subcore_axis_name="s")

    @functools.partial(
        pl.kernel,
        mesh=mesh,
        out_type=jax.ShapeDtypeStruct((B, EMBED), jnp.float32),
        scratch_types=[
            pltpu.VMEM((b_per_w,), jnp.int32),
            pltpu.VMEM((chunk, EMBED), jnp.float32),
            pltpu.VMEM((chunk, EMBED), jnp.float32),
            pltpu.SemaphoreType.DMA,
            pltpu.SemaphoreType.DMA,
            pltpu.SemaphoreType.DMA,
            pltpu.SemaphoreType.DMA,
        ],
        compiler_params=pltpu.CompilerParams(use_tc_tiling_on_sc=False),
    )
    def k(idx_hbm, table_hbm, out_hbm, idx_v, rows0, rows1, sg0, sg1, so0, so1):
        wid = lax.axis_index("s") * NUM_CORES + lax.axis_index("c")
        base = wid * b_per_w
        rows = (rows0, rows1)
        sg = (sg0, sg1)
        so = (so0, so1)
        pltpu.sync_copy(idx_hbm.at[pl.ds(base, b_per_w)], idx_v)

        def gather(c, b):
            return pltpu.async_copy(
                table_hbm.at[idx_v.at[pl.ds(c * chunk, chunk)]], rows[b], sg[b])

        def writeback(c, b):
            return pltpu.async_copy(
                rows[b], out_hbm.at[pl.ds(base + c * chunk, chunk)], so[b])

        g = [None, None]
        w = [None, None]
        g[0] = gather(0, 0)
        for c in range(n_chunks):
            b = c % 2
            if c + 1 < n_chunks:
                if c >= 1:
                    w[1 - b].wait()
                g[1 - b] = gather(c + 1, 1 - b)
            g[b].wait()
            w[b] = writeback(c, b)
        w[(n_chunks - 1) % 2].wait()
        if n_chunks >= 2:
            w[n_chunks % 2].wait()

    return k(idx, table)


def kernel(tokens, embedding_weight):
    B = tokens.shape[0] * tokens.shape[1]
    idx = tokens.reshape(B).astype(jnp.int32)
    b_per_w = B // NUM_WORKERS
    out = _gather_rows(idx, embedding_weight, b_per_w, 1600)
    return out.reshape(tokens.shape + (EMBED,))
